# baseline (speedup 1.0000x reference)
import functools

import jax
import jax.numpy as jnp
from jax import lax
from jax.experimental import pallas as pl
from jax.experimental.pallas import tpu as pltpu
from jax.experimental.pallas import tpu_sc as plsc

N = 10000
NP = N + 8
E = 320000
D = 128

NC = 2
NS = 16
NW = NC * NS
CHUNK = 80
CHUNKS_PER_TILE = 126
EDGES_PER_TILE = E // NW
STRIPE = 640


def _seg_body(g_hbm, packed_hbm, zero_hbm, out_hbm, acc, packed_v,
              r0, r1, r2, sA, sB, sC, sD, sE, sF, dA, dB, dC, dD, dE, dF,
              gs0, gs1, gs2, ss0, ss1, ss2):
    rows = [r0, r1, r2]
    srcs = [sA, sB, sC, sD, sE, sF]
    dsts = [dA, dB, dC, dD, dE, dF]
    gsem = [gs0, gs1, gs2]
    ssem = [ss0, ss1, ss2]

    c = lax.axis_index("c")
    s = lax.axis_index("s")
    wid = s * NC + c
    row_lo = pl.multiple_of(jnp.where(s == NS - 1, NP - STRIPE, s * STRIPE), 8)
    pltpu.async_copy(zero_hbm.at[pl.ds(row_lo, STRIPE)],
                     acc.at[pl.ds(row_lo, STRIPE)], ss0)
    pltpu.sync_copy(packed_hbm.at[wid], packed_v.at[pl.ds(0, CHUNKS_PER_TILE)])
    pltpu.make_async_copy(zero_hbm.at[pl.ds(row_lo, STRIPE)],
                          acc.at[pl.ds(row_lo, STRIPE)], ss0).wait()
    plsc.subcore_barrier()

    def unpack(j, src_b, dst_b):
        for k in range(CHUNK // 16):
            v = packed_v[j, pl.ds(k * 16, 16)]
            src_b[pl.ds(k * 16, 16)] = jnp.bitwise_and(v, 0xFFFF)
            dst_b[pl.ds(k * 16, 16)] = lax.shift_right_logical(v, 16)

    def wait_gather(b):
        pltpu.make_async_copy(g_hbm.at[srcs[0]], rows[b], gsem[b]).wait()

    def wait_scatter(b):
        pltpu.make_async_copy(rows[b], acc.at[srcs[0]], ssem[b]).wait()

    def phase(i, t, do_wait, do_issue):
        b = t % 3
        wait_gather(b)
        pltpu.async_copy(rows[b], acc.at[dsts[t]], ssem[b], add=True)
        if do_wait:
            wait_scatter((b + 2) % 3)
        if do_issue:
            unpack(i + 2, srcs[(t + 2) % 6], dsts[(t + 2) % 6])
            pltpu.async_copy(g_hbm.at[srcs[(t + 2) % 6]],
                             rows[(t + 2) % 3], gsem[(t + 2) % 3])

    unpack(0, srcs[0], dsts[0])
    unpack(1, srcs[1], dsts[1])
    pltpu.async_copy(g_hbm.at[srcs[0]], rows[0], gsem[0])
    pltpu.async_copy(g_hbm.at[srcs[1]], rows[1], gsem[1])

    phase(0, 0, do_wait=False, do_issue=True)
    for t in range(1, 6):
        phase(t, t, do_wait=True, do_issue=True)

    @pl.loop(6, CHUNKS_PER_TILE - 6, step=6)
    def _(j):
        for t in range(6):
            phase(j + t, t, do_wait=True, do_issue=True)

    for t in range(6):
        phase(120 + t, t, do_wait=True, do_issue=(t < 4))
    wait_scatter(125 % 3)

    plsc.subcore_barrier()
    pltpu.sync_copy(acc.at[pl.ds(row_lo, STRIPE)],
                    out_hbm.at[c].at[pl.ds(row_lo, STRIPE)])


_seg_partial = functools.partial(
    pl.kernel,
    out_type=jax.ShapeDtypeStruct((NC, NP, D), jnp.float32),
    mesh=plsc.VectorSubcoreMesh(core_axis_name="c", subcore_axis_name="s"),
    scratch_types=(
        [pltpu.VMEM_SHARED((NP, D), jnp.float32),
         pltpu.VMEM((128, CHUNK), jnp.int32)]
        + [pltpu.VMEM((CHUNK, D), jnp.float32)] * 3
        + [pltpu.VMEM((CHUNK,), jnp.int32)] * 12
        + [pltpu.SemaphoreType.DMA] * 6
    ),
)(_seg_body)


def _tc_first_body(x_ref, wr_ref, wo_ref, b_ref, g_ref, r_ref):
    xv = x_ref[...]
    g_ref[...] = jnp.dot(xv, wr_ref[...], preferred_element_type=jnp.float32)
    r_ref[...] = jnp.dot(xv, wo_ref[...], preferred_element_type=jnp.float32) + b_ref[...]


def _tc_mid_body(p_ref, r_ref, wr_ref, wo_ref, b_ref, g_ref, ro_ref):
    h = jnp.maximum(p_ref[0] + p_ref[1] + r_ref[...], 0.0)
    g_ref[...] = jnp.dot(h, wr_ref[...], preferred_element_type=jnp.float32)
    ro_ref[...] = jnp.dot(h, wo_ref[...], preferred_element_type=jnp.float32) + b_ref[...]


def _tc_last_body(p_ref, r_ref, o_ref):
    o_ref[...] = p_ref[0] + p_ref[1] + r_ref[...]


_nd = jax.ShapeDtypeStruct((NP, D), jnp.float32)

_tc_first = pl.pallas_call(_tc_first_body, out_shape=(_nd, _nd))
_tc_mid = pl.pallas_call(_tc_mid_body, out_shape=(_nd, _nd))
_tc_last = pl.pallas_call(_tc_last_body, out_shape=_nd)


def kernel(x, edge_index, W_rel0, b_rel0, W_root0, W_rel1, b_rel1, W_root1,
           W_rel2, b_rel2, W_root2):
    packed = ((edge_index[1] << 16) | edge_index[0]).reshape(NW, EDGES_PER_TILE)
    pad = jnp.full((NW, CHUNKS_PER_TILE * CHUNK - EDGES_PER_TILE),
                   (N << 16) | N, jnp.int32)
    packed = jnp.concatenate([packed, pad], axis=1).reshape(
        NW, CHUNKS_PER_TILE, CHUNK)
    xp = jnp.concatenate([x, jnp.zeros((NP - N, D), jnp.float32)], axis=0)
    zeros = jnp.zeros((NP, D), jnp.float32)
    params = [(W_rel0, b_rel0, W_root0), (W_rel1, b_rel1, W_root1),
              (W_rel2, b_rel2, W_root2)]

    g, r = _tc_first(xp, W_rel0.T, W_root0.T, b_rel0.reshape(1, D))
    for i in (1, 2):
        p = _seg_partial(g, packed, zeros)
        Wr, br, Wo = params[i]
        g, r = _tc_mid(p, r, Wr.T, Wo.T, br.reshape(1, D))
    p = _seg_partial(g, packed, zeros)
    return _tc_last(p, r)[:N]


# speedup vs baseline: 1.5336x; 1.5336x over previous
import functools

import jax
import jax.numpy as jnp
from jax import lax
from jax.experimental import pallas as pl
from jax.experimental.pallas import tpu as pltpu
from jax.experimental.pallas import tpu_sc as plsc

N = 10000
E = 320000
D = 128

NC = 2
NS = 16
NW = NC * NS
EDGES_PER_TILE = E // NW
CHUNK = 80
CHUNKS_PER_TILE = EDGES_PER_TILE // CHUNK
STRIPE = 640


def _seg_body(g_hbm, packed_hbm, zero_hbm, out_hbm,
              acc, packed_v, rows0, rows1, src0, src1, dst0, dst1,
              gsem0, gsem1, ssem0):
    c = lax.axis_index("c")
    s = lax.axis_index("s")
    wid = s * NC + c
    row_lo = pl.multiple_of(jnp.where(s == NS - 1, N - STRIPE, s * STRIPE), 8)
    pltpu.async_copy(zero_hbm.at[pl.ds(row_lo, STRIPE)],
                     acc.at[pl.ds(row_lo, STRIPE)], ssem0)
    pltpu.sync_copy(packed_hbm.at[wid], packed_v.at[pl.ds(0, CHUNKS_PER_TILE)])
    pltpu.make_async_copy(zero_hbm.at[pl.ds(row_lo, STRIPE)],
                          acc.at[pl.ds(row_lo, STRIPE)], ssem0).wait()
    plsc.subcore_barrier()

    def unpack(j, src_b, dst_b):
        for k in range(CHUNK // 16):
            v = packed_v[j, pl.ds(k * 16, 16)]
            src_b[pl.ds(k * 16, 16)] = jnp.bitwise_and(v, 0xFFFF)
            dst_b[pl.ds(k * 16, 16)] = lax.shift_right_logical(v, 16)

    last = CHUNKS_PER_TILE - 1
    unpack(0, src0, dst0)
    unpack(1, src1, dst1)
    pltpu.async_copy(g_hbm.at[src0], rows0, gsem0)

    @pl.loop(0, last, step=2)
    def _(j):
        pltpu.make_async_copy(g_hbm.at[src0], rows0, gsem0).wait()
        pltpu.async_copy(g_hbm.at[src1], rows1, gsem1)
        pltpu.sync_copy(rows0, acc.at[dst0], add=True)
        unpack(j + 2, src0, dst0)
        pltpu.async_copy(g_hbm.at[src0], rows0, gsem0)
        pltpu.make_async_copy(g_hbm.at[src1], rows1, gsem1).wait()
        pltpu.sync_copy(rows1, acc.at[dst1], add=True)
        unpack(j + 3, src1, dst1)

    pltpu.make_async_copy(g_hbm.at[src0], rows0, gsem0).wait()
    pltpu.sync_copy(rows0, acc.at[dst0], add=True)

    plsc.subcore_barrier()
    pltpu.sync_copy(acc.at[pl.ds(row_lo, STRIPE)],
                    out_hbm.at[c].at[pl.ds(row_lo, STRIPE)])


_seg_partial = functools.partial(
    pl.kernel,
    out_type=jax.ShapeDtypeStruct((NC, N, D), jnp.float32),
    mesh=plsc.VectorSubcoreMesh(core_axis_name="c", subcore_axis_name="s"),
    scratch_types=[
        pltpu.VMEM_SHARED((N, D), jnp.float32),
        pltpu.VMEM((128, CHUNK), jnp.int32),
        pltpu.VMEM((CHUNK, D), jnp.float32),
        pltpu.VMEM((CHUNK, D), jnp.float32),
        pltpu.VMEM((CHUNK,), jnp.int32),
        pltpu.VMEM((CHUNK,), jnp.int32),
        pltpu.VMEM((CHUNK,), jnp.int32),
        pltpu.VMEM((CHUNK,), jnp.int32),
        pltpu.SemaphoreType.DMA,
        pltpu.SemaphoreType.DMA,
        pltpu.SemaphoreType.DMA,
    ],
)(_seg_body)


def _tc_first_body(x_ref, wr_ref, wo_ref, b_ref, g_ref, r_ref):
    xv = x_ref[...]
    g_ref[...] = jnp.dot(xv, wr_ref[...], preferred_element_type=jnp.float32)
    r_ref[...] = jnp.dot(xv, wo_ref[...], preferred_element_type=jnp.float32) + b_ref[...]


def _tc_mid_body(p_ref, r_ref, wr_ref, wo_ref, b_ref, g_ref, ro_ref):
    h = jnp.maximum(p_ref[0] + p_ref[1] + r_ref[...], 0.0)
    g_ref[...] = jnp.dot(h, wr_ref[...], preferred_element_type=jnp.float32)
    ro_ref[...] = jnp.dot(h, wo_ref[...], preferred_element_type=jnp.float32) + b_ref[...]


def _tc_last_body(p_ref, r_ref, o_ref):
    o_ref[...] = p_ref[0] + p_ref[1] + r_ref[...]


_nd = jax.ShapeDtypeStruct((N, D), jnp.float32)

_tc_first = pl.pallas_call(_tc_first_body, out_shape=(_nd, _nd))
_tc_mid = pl.pallas_call(_tc_mid_body, out_shape=(_nd, _nd))
_tc_last = pl.pallas_call(_tc_last_body, out_shape=_nd)


def kernel(x, edge_index, W_rel0, b_rel0, W_root0, W_rel1, b_rel1, W_root1,
           W_rel2, b_rel2, W_root2):
    packed = ((edge_index[1] << 16) | edge_index[0]).reshape(
        NW, CHUNKS_PER_TILE, CHUNK)
    zeros = jnp.zeros((N, D), jnp.float32)
    params = [(W_rel0, b_rel0, W_root0), (W_rel1, b_rel1, W_root1),
              (W_rel2, b_rel2, W_root2)]

    g, r = _tc_first(x, W_rel0.T, W_root0.T, b_rel0.reshape(1, D))
    for i in (1, 2):
        p = _seg_partial(g, packed, zeros)
        Wr, br, Wo = params[i]
        g, r = _tc_mid(p, r, Wr.T, Wo.T, br.reshape(1, D))
    p = _seg_partial(g, packed, zeros)
    return _tc_last(p, r)


# speedup vs baseline: 1.5453x; 1.0076x over previous
import functools

import jax
import jax.numpy as jnp
from jax import lax
from jax.experimental import pallas as pl
from jax.experimental.pallas import tpu as pltpu
from jax.experimental.pallas import tpu_sc as plsc

N = 10000
E = 320000
D = 128

NC = 2
NS = 16
NW = NC * NS
EDGES_PER_TILE = E // NW
CHUNK = 80
CHUNKS_PER_TILE = EDGES_PER_TILE // CHUNK
STRIPE = 640


def _seg_body(g_hbm, packed_hbm, zero_hbm, out_hbm,
              acc, packed_v, rows0, rows1, src0, src1, dst0, dst1,
              gsem0, gsem1, ssem0):
    c = lax.axis_index("c")
    s = lax.axis_index("s")
    wid = s * NC + c
    row_lo = pl.multiple_of(jnp.where(s == NS - 1, N - STRIPE, s * STRIPE), 8)
    pltpu.async_copy(zero_hbm.at[pl.ds(row_lo, STRIPE)],
                     acc.at[pl.ds(row_lo, STRIPE)], ssem0)
    pltpu.sync_copy(packed_hbm.at[wid], packed_v.at[pl.ds(0, CHUNKS_PER_TILE)])
    def unpack(j, src_b, dst_b):
        for k in range(CHUNK // 16):
            v = packed_v[j, pl.ds(k * 16, 16)]
            src_b[pl.ds(k * 16, 16)] = jnp.bitwise_and(v, 0xFFFF)
            dst_b[pl.ds(k * 16, 16)] = lax.shift_right_logical(v, 16)

    last = CHUNKS_PER_TILE - 1
    unpack(0, src0, dst0)
    unpack(1, src1, dst1)
    pltpu.async_copy(g_hbm.at[src0], rows0, gsem0)
    pltpu.make_async_copy(zero_hbm.at[pl.ds(row_lo, STRIPE)],
                          acc.at[pl.ds(row_lo, STRIPE)], ssem0).wait()
    plsc.subcore_barrier()

    @pl.loop(0, last, step=2)
    def _(j):
        pltpu.make_async_copy(g_hbm.at[src0], rows0, gsem0).wait()
        pltpu.async_copy(g_hbm.at[src1], rows1, gsem1)
        pltpu.sync_copy(rows0, acc.at[dst0], add=True)
        unpack(j + 2, src0, dst0)
        pltpu.async_copy(g_hbm.at[src0], rows0, gsem0)
        pltpu.make_async_copy(g_hbm.at[src1], rows1, gsem1).wait()
        pltpu.sync_copy(rows1, acc.at[dst1], add=True)
        unpack(j + 3, src1, dst1)

    pltpu.make_async_copy(g_hbm.at[src0], rows0, gsem0).wait()
    pltpu.sync_copy(rows0, acc.at[dst0], add=True)

    plsc.subcore_barrier()
    pltpu.sync_copy(acc.at[pl.ds(row_lo, STRIPE)],
                    out_hbm.at[c].at[pl.ds(row_lo, STRIPE)])


_seg_partial = functools.partial(
    pl.kernel,
    out_type=jax.ShapeDtypeStruct((NC, N, D), jnp.float32),
    mesh=plsc.VectorSubcoreMesh(core_axis_name="c", subcore_axis_name="s"),
    scratch_types=[
        pltpu.VMEM_SHARED((N, D), jnp.float32),
        pltpu.VMEM((128, CHUNK), jnp.int32),
        pltpu.VMEM((CHUNK, D), jnp.float32),
        pltpu.VMEM((CHUNK, D), jnp.float32),
        pltpu.VMEM((CHUNK,), jnp.int32),
        pltpu.VMEM((CHUNK,), jnp.int32),
        pltpu.VMEM((CHUNK,), jnp.int32),
        pltpu.VMEM((CHUNK,), jnp.int32),
        pltpu.SemaphoreType.DMA,
        pltpu.SemaphoreType.DMA,
        pltpu.SemaphoreType.DMA,
    ],
)(_seg_body)


def _tc_first_body(x_ref, wr_ref, wo_ref, b_ref, g_ref, r_ref):
    xv = x_ref[...]
    g_ref[...] = jnp.dot(xv, wr_ref[...], preferred_element_type=jnp.float32)
    r_ref[...] = jnp.dot(xv, wo_ref[...], preferred_element_type=jnp.float32) + b_ref[...]


def _tc_mid_body(p_ref, r_ref, wr_ref, wo_ref, b_ref, g_ref, ro_ref):
    h = jnp.maximum(p_ref[0] + p_ref[1] + r_ref[...], 0.0)
    g_ref[...] = jnp.dot(h, wr_ref[...], preferred_element_type=jnp.float32)
    ro_ref[...] = jnp.dot(h, wo_ref[...], preferred_element_type=jnp.float32) + b_ref[...]


def _tc_last_body(p_ref, r_ref, o_ref):
    o_ref[...] = p_ref[0] + p_ref[1] + r_ref[...]


_nd = jax.ShapeDtypeStruct((N, D), jnp.float32)

_tc_first = pl.pallas_call(_tc_first_body, out_shape=(_nd, _nd))
_tc_mid = pl.pallas_call(_tc_mid_body, out_shape=(_nd, _nd))
_tc_last = pl.pallas_call(_tc_last_body, out_shape=_nd)


def kernel(x, edge_index, W_rel0, b_rel0, W_root0, W_rel1, b_rel1, W_root1,
           W_rel2, b_rel2, W_root2):
    packed = ((edge_index[1] << 16) | edge_index[0]).reshape(
        NW, CHUNKS_PER_TILE, CHUNK)
    zeros = jnp.zeros((N, D), jnp.float32)
    params = [(W_rel0, b_rel0, W_root0), (W_rel1, b_rel1, W_root1),
              (W_rel2, b_rel2, W_root2)]

    g, r = _tc_first(x, W_rel0.T, W_root0.T, b_rel0.reshape(1, D))
    for i in (1, 2):
        p = _seg_partial(g, packed, zeros)
        Wr, br, Wo = params[i]
        g, r = _tc_mid(p, r, Wr.T, Wo.T, br.reshape(1, D))
    p = _seg_partial(g, packed, zeros)
    return _tc_last(p, r)
